# baseline (speedup 1.0000x reference)
import functools

import jax
import jax.numpy as jnp
from jax import lax
from jax.experimental import pallas as pl
from jax.experimental.pallas import tpu as pltpu
from jax.experimental.pallas import tpu_sc as plsc

D = 64
SCALE = 8.0
IPG = 128
CHUNK = 1024
GATHERS = CHUNK // IPG


def _body(idx_hbm, table_hbm, out_hbm, idx_v, rows_v, sem, *, b_per_w, nc):
    wid = lax.axis_index("s") * nc + lax.axis_index("c")
    base = wid * b_per_w
    n_chunks = b_per_w // CHUNK

    def chunk_body(j, carry):
        off = base + j * CHUNK
        row_off = pl.multiple_of(off // IPG, 8)
        pltpu.sync_copy(idx_hbm.at[pl.ds(row_off, GATHERS)], idx_v)
        copies = []
        for g in range(GATHERS):
            copies.append(
                pltpu.async_copy(
                    table_hbm.at[idx_v.at[g]],
                    rows_v.at[pl.ds(g * IPG, IPG)],
                    sem,
                )
            )
        for c in copies:
            c.wait()

        def scale_body(r, carry2):
            for k in range(D // 16):
                sl = (r, pl.ds(k * 16, 16))
                rows_v[sl] = rows_v[sl] * SCALE
            return carry2

        lax.fori_loop(0, CHUNK, scale_body, 0, unroll=4)

        pltpu.sync_copy(rows_v, out_hbm.at[pl.ds(off, CHUNK)])
        return carry

    lax.fori_loop(0, n_chunks, chunk_body, 0)


def kernel(tokens, table):
    batch, hist = tokens.shape
    B = batch * hist
    info = plsc.get_sparse_core_info()
    nc, ns = info.num_cores, info.num_subcores
    nw = nc * ns
    b_per_w = B // nw
    assert b_per_w % CHUNK == 0

    idx = tokens.reshape(B // IPG, IPG).astype(jnp.int32)

    mesh = plsc.VectorSubcoreMesh(core_axis_name="c", subcore_axis_name="s")

    run = pl.kernel(
        functools.partial(_body, b_per_w=b_per_w, nc=nc),
        mesh=mesh,
        out_type=jax.ShapeDtypeStruct((B, D), jnp.float32),
        scratch_types=[
            pltpu.VMEM((GATHERS, IPG), jnp.int32),
            pltpu.VMEM((CHUNK, D), jnp.float32),
            pltpu.SemaphoreType.DMA,
        ],
        compiler_params=pltpu.CompilerParams(use_tc_tiling_on_sc=False),
    )
    out = run(idx, table)
    return out.reshape(batch, hist, D)
